# baseline (speedup 1.0000x reference)
import functools

import jax
import jax.numpy as jnp
from jax import lax
from jax.experimental import pallas as pl
from jax.experimental.pallas import tpu as pltpu
from jax.experimental.pallas import tpu_sc as plsc

f32 = jnp.float32
bf16 = jnp.bfloat16

NC = 2
NS = 16
NW = NC * NS
CH = 128


def _dot(a, b):
    return jnp.dot(a, b, preferred_element_type=f32)


def _bd(W, P, in_off, out_off, shape):
    M = jnp.zeros(shape, f32)
    for g in range(P):
        M = M.at[in_off * g:in_off * g + W.shape[0],
                 out_off * g:out_off * g + W.shape[1]].set(W)
    return M


def _enc_node(x, W1, b1, W2, b2):
    M, K = x.shape
    H, O = W1.shape[1], W2.shape[1]
    BR = 2000

    def body(h_ref, w1_ref, b1_ref, w2_ref, b2_ref, o_ref):
        a = jnp.maximum(_dot(h_ref[...].astype(bf16), w1_ref[...])
                        + b1_ref[...], 0.0)
        o_ref[...] = jnp.maximum(_dot(a.astype(bf16), w2_ref[...])
                                 + b2_ref[...], 0.0).astype(bf16)

    return pl.pallas_call(
        body,
        grid=(M // BR,),
        in_specs=[
            pl.BlockSpec((BR, K), lambda i: (i, 0)),
            pl.BlockSpec((K, H), lambda i: (0, 0)),
            pl.BlockSpec((1, H), lambda i: (0, 0)),
            pl.BlockSpec((H, O), lambda i: (0, 0)),
            pl.BlockSpec((1, O), lambda i: (0, 0)),
        ],
        out_specs=pl.BlockSpec((BR, O), lambda i: (i, 0)),
        out_shape=jax.ShapeDtypeStruct((M, O), bf16),
    )(x, W1.astype(bf16), b1.reshape(1, -1), W2.astype(bf16),
      b2.reshape(1, -1))


def _enc_edge8(a8, Wa1_8, ba1_8, Wa2_8, ba2_8):
    M = a8.shape[0]
    BQ = 1000

    def body(a_ref, w1_ref, b1_ref, w2_ref, b2_ref, o_ref):
        h = jnp.maximum(_dot(a_ref[...].astype(bf16), w1_ref[...])
                        + b1_ref[...], 0.0)
        o_ref[...] = jnp.maximum(_dot(h.astype(bf16), w2_ref[...])
                                 + b2_ref[...], 0.0).astype(bf16)

    return pl.pallas_call(
        body,
        grid=(M // BQ,),
        in_specs=[
            pl.BlockSpec((BQ, 128), lambda i: (i, 0)),
            pl.BlockSpec((128, 144), lambda i: (0, 0)),
            pl.BlockSpec((1, 144), lambda i: (0, 0)),
            pl.BlockSpec((144, 256), lambda i: (0, 0)),
            pl.BlockSpec((1, 256), lambda i: (0, 0)),
        ],
        out_specs=pl.BlockSpec((BQ, 256), lambda i: (i, 0)),
        out_shape=jax.ShapeDtypeStruct((M, 256), bf16),
    )(a8, Wa1_8.astype(bf16), ba1_8, Wa2_8.astype(bf16), ba2_8)


def _edge_step4(sl4, tl4, leie4, wp):
    M = sl4.shape[0]
    BP = 1000

    def body(sl_r, tl_r, leie_r, w4e, b4e, w2p, b2p, iem,
             wf4, bf4, wcfi, bcfi, wcfo, bcfo,
             leie_o, fi_o, fo_o):
        leie_b = leie_r[...]
        e_in = jnp.concatenate([sl_r[...], tl_r[...], leie_b], axis=1)
        h1 = jnp.maximum(_dot(e_in, w4e[...]) + b4e[...], 0.0)
        leie_n = (jnp.maximum(_dot(h1.astype(bf16), w2p[...]) + b2p[...],
                              0.0)
                  + leie_b.astype(f32) * iem[...]).astype(bf16)
        leie_o[...] = leie_n
        f_in = jnp.concatenate([sl_r[...], leie_n], axis=1)
        h2 = jnp.maximum(_dot(f_in, wf4[...]) + bf4[...], 0.0)
        h2b = h2.astype(bf16)
        fi_o[...] = jnp.maximum(_dot(h2b, wcfi[...]) + bcfi[...], 0.0)
        fo_o[...] = jnp.maximum(_dot(h2b, wcfo[...]) + bcfo[...], 0.0)

    full = lambda a: pl.BlockSpec(a.shape, lambda i: (0,) * a.ndim)
    return pl.pallas_call(
        body,
        grid=(M // BP,),
        in_specs=[
            pl.BlockSpec((BP, 128), lambda i: (i, 0)),
            pl.BlockSpec((BP, 128), lambda i: (i, 0)),
            pl.BlockSpec((BP, 128), lambda i: (i, 0)),
        ] + [full(wp[k]) for k in
             ("w4e", "b4e", "w2p", "b2p", "iem",
              "wf4", "bf4", "wcfi", "bcfi", "wcfo", "bcfo")],
        out_specs=[
            pl.BlockSpec((BP, 128), lambda i: (i, 0)),
            pl.BlockSpec((BP, 128), lambda i: (i, 0)),
            pl.BlockSpec((BP, 128), lambda i: (i, 0)),
        ],
        out_shape=[
            jax.ShapeDtypeStruct((M, 128), bf16),
            jax.ShapeDtypeStruct((M, 128), f32),
            jax.ShapeDtypeStruct((M, 128), f32),
        ],
    )(sl4, tl4, leie4,
      wp["w4e"], wp["b4e"], wp["w2p"], wp["b2p"], wp["iem"],
      wp["wf4"], wp["bf4"], wp["wcfi"], wp["bcfi"], wp["wcfo"], wp["bcfo"])


def _edge_final4(sl4, tl4, leie4, wp):
    M = sl4.shape[0]
    BP = 1000

    def body(sl_r, tl_r, leie_r, w4e, b4e, w2c, b2c, wc14, bc14, wc24, bc24,
             o_ref):
        e_in = jnp.concatenate([sl_r[...], tl_r[...], leie_r[...]], axis=1)
        h1 = jnp.maximum(_dot(e_in, w4e[...]) + b4e[...], 0.0)
        le_n = jnp.maximum(_dot(h1.astype(bf16), w2c[...]) + b2c[...], 0.0)
        hc = jnp.maximum(_dot(le_n.astype(bf16), wc14[...]) + bc14[...], 0.0)
        o_ref[...] = _dot(hc.astype(bf16), wc24[...]) + bc24[...]

    full = lambda a: pl.BlockSpec(a.shape, lambda i: (0,) * a.ndim)
    return pl.pallas_call(
        body,
        grid=(M // BP,),
        in_specs=[
            pl.BlockSpec((BP, 128), lambda i: (i, 0)),
            pl.BlockSpec((BP, 128), lambda i: (i, 0)),
            pl.BlockSpec((BP, 128), lambda i: (i, 0)),
        ] + [full(wp[k]) for k in
             ("w4e", "b4e", "w2c", "b2c", "wc14", "bc14", "wc24", "bc24")],
        out_specs=pl.BlockSpec((BP, 4), lambda i: (i, 0)),
        out_shape=jax.ShapeDtypeStruct((M, 4), f32),
    )(sl4, tl4, leie4,
      wp["w4e"], wp["b4e"], wp["w2c"], wp["b2c"],
      wp["wc14"], wp["bc14"], wp["wc24"], wp["bc24"])


def _node_mlp4(pfo4, pfi4, wnfi4, wnfo4, bn4, n):
    NA = pfo4.shape[1]
    NP = n // 4

    def body(pfo_r, pfi_r, wi_ref, wo_ref, b_ref, o_ref):
        pfo = (pfo_r[0, :NP] + pfo_r[1, :NP]).astype(bf16)
        pfi = (pfi_r[0, :NP] + pfi_r[1, :NP]).astype(bf16)
        o_ref[...] = jnp.maximum(
            _dot(pfi, wi_ref[...]) + _dot(pfo, wo_ref[...]) + b_ref[...],
            0.0).astype(bf16)

    return pl.pallas_call(
        body,
        out_shape=jax.ShapeDtypeStruct((NP, 128), bf16),
    )(pfo4, pfi4, wnfi4.astype(bf16), wnfo4.astype(bf16), bn4)


KSUP = 4


def _sc_gather(lx_lin, row2d, col2d):
    nr = row2d.shape[0]
    n, D = lx_lin.shape
    assert nr % KSUP == 0
    nsup = nr // KSUP
    per_w = (nsup + NW - 1) // NW
    mesh = plsc.VectorSubcoreMesh(core_axis_name="c", subcore_axis_name="s")

    @functools.partial(
        pl.kernel,
        out_type=(jax.ShapeDtypeStruct((nr, CH, D), bf16),
                  jax.ShapeDtypeStruct((nr, CH, D), bf16)),
        mesh=mesh,
        scratch_types=[
            pltpu.VMEM((KSUP, CH), jnp.int32),
            pltpu.VMEM((KSUP, CH), jnp.int32),
            pltpu.VMEM((KSUP, CH, D), bf16),
            pltpu.VMEM((KSUP, CH, D), bf16),
            pltpu.SemaphoreType.DMA,
        ],
        compiler_params=pltpu.CompilerParams(use_tc_tiling_on_sc=False),
    )
    def k(lx_hbm, row_hbm, col_hbm, sl_hbm, tl_hbm, ri, ci, rbuf, cbuf, sem):
        wid = lax.axis_index("s") * NC + lax.axis_index("c")

        def body(j, carry):
            s = wid + j * NW

            @pl.when(s < nsup)
            def _():
                base = pl.multiple_of(s * KSUP, KSUP)
                pltpu.sync_copy(row_hbm.at[pl.ds(base, KSUP)], ri)
                pltpu.sync_copy(col_hbm.at[pl.ds(base, KSUP)], ci)
                cps = []
                for jj in range(KSUP):
                    cps.append(pltpu.async_copy(
                        lx_hbm.at[ri.at[jj]], rbuf.at[jj], sem))
                    cps.append(pltpu.async_copy(
                        lx_hbm.at[ci.at[jj]], cbuf.at[jj], sem))
                for cp in cps:
                    cp.wait()
                pltpu.sync_copy(rbuf, sl_hbm.at[pl.ds(base, KSUP)])
                pltpu.sync_copy(cbuf, tl_hbm.at[pl.ds(base, KSUP)])
            return carry

        lax.fori_loop(0, per_w, body, 0)

    return k(lx_lin, row2d, col2d)


def _sc_scatter2(fo3, fi3, ifo2d, ifi2d, zeros):
    nr, _, D = fo3.shape
    n_acc = zeros.shape[0]
    assert nr % KSUP == 0 and n_acc % NS == 0
    nsup = nr // KSUP
    per_w = (nsup + NW - 1) // NW
    rpt = n_acc // NS
    mesh = plsc.VectorSubcoreMesh(core_axis_name="c", subcore_axis_name="s")

    @functools.partial(
        pl.kernel,
        out_type=(jax.ShapeDtypeStruct((NC, n_acc, D), f32),
                  jax.ShapeDtypeStruct((NC, n_acc, D), f32)),
        mesh=mesh,
        scratch_types=[
            pltpu.VMEM_SHARED((n_acc, D), f32),
            pltpu.VMEM_SHARED((n_acc, D), f32),
            pltpu.VMEM((KSUP, CH), jnp.int32),
            pltpu.VMEM((KSUP, CH), jnp.int32),
            pltpu.VMEM((KSUP, CH, D), f32),
            pltpu.VMEM((KSUP, CH, D), f32),
        ],
        compiler_params=pltpu.CompilerParams(use_tc_tiling_on_sc=False),
    )
    def k(fo_hbm, fi_hbm, ifo_hbm, ifi_hbm, zeros_hbm, out_fo, out_fi,
          acc_fo, acc_fi, ri, si, dbuf, ebuf):
        cid = lax.axis_index("c")
        sid = lax.axis_index("s")
        wid = sid * NC + cid

        @pl.when(sid == 0)
        def _():
            pltpu.sync_copy(zeros_hbm, acc_fo)
            pltpu.sync_copy(zeros_hbm, acc_fi)

        plsc.subcore_barrier()

        def body(j, carry):
            s = wid + j * NW

            @pl.when(s < nsup)
            def _():
                base = pl.multiple_of(s * KSUP, KSUP)
                pltpu.sync_copy(ifo_hbm.at[pl.ds(base, KSUP)], ri)
                pltpu.sync_copy(ifi_hbm.at[pl.ds(base, KSUP)], si)
                pltpu.sync_copy(fo_hbm.at[pl.ds(base, KSUP)], dbuf)
                pltpu.sync_copy(fi_hbm.at[pl.ds(base, KSUP)], ebuf)
                for jj in range(KSUP):
                    pltpu.sync_copy(dbuf.at[jj], acc_fo.at[ri.at[jj]],
                                    add=True)
                    pltpu.sync_copy(ebuf.at[jj], acc_fi.at[si.at[jj]],
                                    add=True)
            return carry

        lax.fori_loop(0, per_w, body, 0)
        plsc.subcore_barrier()
        pltpu.sync_copy(acc_fo.at[pl.ds(sid * rpt, rpt)],
                        out_fo.at[cid].at[pl.ds(sid * rpt, rpt)])
        pltpu.sync_copy(acc_fi.at[pl.ds(sid * rpt, rpt)],
                        out_fi.at[cid].at[pl.ds(sid * rpt, rpt)])

    return k(fo3, fi3, ifo2d, ifi2d, zeros)


def kernel(x, edge_index, edge_attr, params):
    n = x.shape[0]
    E = edge_index.shape[1]
    row = edge_index[0]
    col = edge_index[1]
    p = params

    idx_fo = jnp.where(row < col, row, n).reshape(E // CH, CH)
    idx_fi = jnp.where(row > col, row, n).reshape(E // CH, CH)
    row2d = row.reshape(E // CH, CH)
    col2d = col.reshape(E // CH, CH)
    n_acc = n + 16

    (W1, b1), (W2, b2) = p["edge_model"]
    (Wfo1, bfo1), (Wfo2, bfo2) = p["flow_out"]
    (Wfi1, bfi1), (Wfi2, bfi2) = p["flow_in"]
    Wf1 = jnp.concatenate([Wfo1, Wfi1], axis=1)
    bf1c = jnp.concatenate([bfo1, bfi1])
    (Wn_, bn_) = p["node_mlp"][0]
    (Wc1, bc1), (Wc2, bc2) = p["classifier"]

    w4e = jnp.zeros((384, 320), f32)
    w2p = jnp.zeros((320, 128), f32)
    b2p = jnp.zeros((1, 128), f32)
    iem = jnp.zeros((1, 128), f32)
    wf4 = jnp.zeros((256, 448), f32)
    wcfi = jnp.zeros((448, 128), f32)
    wcfo = jnp.zeros((448, 128), f32)
    for g in range(4):
        w4e = w4e.at[32 * g:32 * g + 32, 80 * g:80 * g + 80].set(W1[:32])
        w4e = w4e.at[128 + 32 * g:128 + 32 * g + 32,
                     80 * g:80 * g + 80].set(W1[32:64])
        w4e = w4e.at[256 + 32 * g:256 + 32 * g + 32,
                     80 * g:80 * g + 80].set(W1[64:96])
        w2p = w2p.at[80 * g:80 * g + 80, 32 * g:32 * g + 16].set(W2)
        b2p = b2p.at[0, 32 * g:32 * g + 16].set(b2)
        iem = iem.at[0, 32 * g + 16:32 * g + 32].set(1.0)
        wf4 = wf4.at[32 * g:32 * g + 32, 112 * g:112 * g + 112].set(Wf1[:32])
        wf4 = wf4.at[128 + 32 * g:128 + 32 * g + 16,
                     112 * g:112 * g + 112].set(Wf1[32:48])
        wcfo = wcfo.at[112 * g:112 * g + 56,
                       32 * g:32 * g + 32].set(Wfo2)
        wcfi = wcfi.at[112 * g + 56:112 * g + 112,
                       32 * g:32 * g + 32].set(Wfi2)
    wp = {
        "w4e": w4e.astype(bf16), "b4e": jnp.tile(b1, 4).reshape(1, 320),
        "w2p": w2p.astype(bf16), "b2p": b2p, "iem": iem,
        "wf4": wf4.astype(bf16), "bf4": jnp.tile(bf1c, 4).reshape(1, 448),
        "wcfi": wcfi.astype(bf16), "bcfi": jnp.tile(bfi2, 4).reshape(1, 128),
        "wcfo": wcfo.astype(bf16), "bcfo": jnp.tile(bfo2, 4).reshape(1, 128),
        "w2c": _bd(W2, 4, 80, 16, (320, 64)).astype(bf16),
        "b2c": jnp.tile(b2, 4).reshape(1, 64),
        "wc14": _bd(Wc1, 4, 16, 8, (64, 32)).astype(bf16),
        "bc14": jnp.tile(bc1, 4).reshape(1, 32),
        "wc24": _bd(Wc2, 4, 8, 1, (32, 4)).astype(bf16),
        "bc24": jnp.tile(bc2, 4).reshape(1, 4),
    }
    wnfi4 = _bd(Wn_[:32], 4, 32, 32, (128, 128))
    wnfo4 = _bd(Wn_[32:], 4, 32, 32, (128, 128))
    bn4 = jnp.tile(bn_, 4).reshape(1, 128)

    (Wa1, ba1), (Wa2, ba2) = p["enc_edge"]
    wa1_8 = _bd(Wa1, 8, 16, 18, (128, 144))
    ba1_8 = jnp.tile(ba1, 8).reshape(1, 144)
    wa2_8 = jnp.zeros((144, 256), f32)
    ba2_8 = jnp.zeros((1, 256), f32)
    for g in range(8):
        wa2_8 = wa2_8.at[18 * g:18 * g + 18, 32 * g:32 * g + 16].set(Wa2)
        wa2_8 = wa2_8.at[18 * g:18 * g + 18,
                         32 * g + 16:32 * g + 32].set(Wa2)
        ba2_8 = ba2_8.at[0, 32 * g:32 * g + 16].set(ba2)
        ba2_8 = ba2_8.at[0, 32 * g + 16:32 * g + 32].set(ba2)

    (We1, be1), (We2, be2) = p["enc_node"]
    lx = _enc_node(x, We1, be1, We2, be2)
    lx4 = lx.reshape(n // 4, 128)
    leie8 = _enc_edge8(edge_attr.reshape(E // 8, 128),
                       wa1_8, ba1_8, wa2_8, ba2_8)
    leie4 = leie8.reshape(E // 4, 128)

    zeros = jnp.zeros((n_acc, 32), f32)
    for step in range(1, 5):
        sl, tl = _sc_gather(lx4.reshape(n, 32), row2d, col2d)
        sl4 = sl.reshape(E // 4, 128)
        tl4 = tl.reshape(E // 4, 128)
        if step == 4:
            logits4 = _edge_final4(sl4, tl4, leie4, wp)
            return logits4.reshape(E, 1)
        leie4, fi4, fo4 = _edge_step4(sl4, tl4, leie4, wp)
        pfo, pfi = _sc_scatter2(fo4.reshape(E // CH, CH, 32),
                                fi4.reshape(E // CH, CH, 32),
                                idx_fo, idx_fi, zeros)
        lx4 = _node_mlp4(pfo.reshape(NC, n_acc // 4, 128),
                         pfi.reshape(NC, n_acc // 4, 128),
                         wnfi4, wnfo4, bn4, n)


# speedup vs baseline: 1.5493x; 1.5493x over previous
import functools

import jax
import jax.numpy as jnp
from jax import lax
from jax.experimental import pallas as pl
from jax.experimental.pallas import tpu as pltpu
from jax.experimental.pallas import tpu_sc as plsc

f32 = jnp.float32
bf16 = jnp.bfloat16

NC = 2
NS = 16
NW = NC * NS
CH = 128


def _dot(a, b):
    return jnp.dot(a, b, preferred_element_type=f32)


def _bd(W, P, in_off, out_off, shape):
    M = jnp.zeros(shape, f32)
    for g in range(P):
        M = M.at[in_off * g:in_off * g + W.shape[0],
                 out_off * g:out_off * g + W.shape[1]].set(W)
    return M


def _enc_node(x, W1, b1, W2, b2):
    M, K = x.shape
    H, O = W1.shape[1], W2.shape[1]
    BR = 2000

    def body(h_ref, w1_ref, b1_ref, w2_ref, b2_ref, o_ref):
        a = jnp.maximum(_dot(h_ref[...].astype(bf16), w1_ref[...])
                        + b1_ref[...], 0.0)
        o_ref[...] = jnp.maximum(_dot(a.astype(bf16), w2_ref[...])
                                 + b2_ref[...], 0.0)

    return pl.pallas_call(
        body,
        grid=(M // BR,),
        in_specs=[
            pl.BlockSpec((BR, K), lambda i: (i, 0)),
            pl.BlockSpec((K, H), lambda i: (0, 0)),
            pl.BlockSpec((1, H), lambda i: (0, 0)),
            pl.BlockSpec((H, O), lambda i: (0, 0)),
            pl.BlockSpec((1, O), lambda i: (0, 0)),
        ],
        out_specs=pl.BlockSpec((BR, O), lambda i: (i, 0)),
        out_shape=jax.ShapeDtypeStruct((M, O), f32),
    )(x, W1.astype(bf16), b1.reshape(1, -1), W2.astype(bf16),
      b2.reshape(1, -1))


def _enc_edge8(a8, Wa1_8, ba1_8, Wa2_8, ba2_8):
    M = a8.shape[0]
    BQ = 1000

    def body(a_ref, w1_ref, b1_ref, w2_ref, b2_ref, o_ref):
        h = jnp.maximum(_dot(a_ref[...].astype(bf16), w1_ref[...])
                        + b1_ref[...], 0.0)
        o_ref[...] = jnp.maximum(_dot(h.astype(bf16), w2_ref[...])
                                 + b2_ref[...], 0.0)

    return pl.pallas_call(
        body,
        grid=(M // BQ,),
        in_specs=[
            pl.BlockSpec((BQ, 128), lambda i: (i, 0)),
            pl.BlockSpec((128, 144), lambda i: (0, 0)),
            pl.BlockSpec((1, 144), lambda i: (0, 0)),
            pl.BlockSpec((144, 256), lambda i: (0, 0)),
            pl.BlockSpec((1, 256), lambda i: (0, 0)),
        ],
        out_specs=pl.BlockSpec((BQ, 256), lambda i: (i, 0)),
        out_shape=jax.ShapeDtypeStruct((M, 256), f32),
    )(a8, Wa1_8.astype(bf16), ba1_8, Wa2_8.astype(bf16), ba2_8)


def _edge_step4(sl4, tl4, leie4, wp):
    M = sl4.shape[0]
    BP = 800

    def body(sl_r, tl_r, leie_r, w4e, b4e, w2p, b2p, iem,
             wf4, bf4, wcfi, bcfi, wcfo, bcfo,
             leie_o, fi_o, fo_o):
        leie_f32 = leie_r[...]
        e_in = jnp.concatenate(
            [sl_r[...].astype(bf16), tl_r[...].astype(bf16),
             leie_f32.astype(bf16)], axis=1)
        h1 = jnp.maximum(_dot(e_in, w4e[...]) + b4e[...], 0.0)
        leie_n = (jnp.maximum(_dot(h1.astype(bf16), w2p[...]) + b2p[...],
                              0.0)
                  + leie_f32 * iem[...])
        leie_o[...] = leie_n
        f_in = jnp.concatenate(
            [sl_r[...].astype(bf16), leie_n.astype(bf16)], axis=1)
        h2 = jnp.maximum(_dot(f_in, wf4[...]) + bf4[...], 0.0)
        h2b = h2.astype(bf16)
        fi_o[...] = jnp.maximum(_dot(h2b, wcfi[...]) + bcfi[...], 0.0)
        fo_o[...] = jnp.maximum(_dot(h2b, wcfo[...]) + bcfo[...], 0.0)

    full = lambda a: pl.BlockSpec(a.shape, lambda i: (0,) * a.ndim)
    return pl.pallas_call(
        body,
        grid=(M // BP,),
        in_specs=[
            pl.BlockSpec((BP, 128), lambda i: (i, 0)),
            pl.BlockSpec((BP, 128), lambda i: (i, 0)),
            pl.BlockSpec((BP, 128), lambda i: (i, 0)),
        ] + [full(wp[k]) for k in
             ("w4e", "b4e", "w2p", "b2p", "iem",
              "wf4", "bf4", "wcfi", "bcfi", "wcfo", "bcfo")],
        out_specs=[
            pl.BlockSpec((BP, 128), lambda i: (i, 0)),
            pl.BlockSpec((BP, 128), lambda i: (i, 0)),
            pl.BlockSpec((BP, 128), lambda i: (i, 0)),
        ],
        out_shape=[
            jax.ShapeDtypeStruct((M, 128), f32),
            jax.ShapeDtypeStruct((M, 128), f32),
            jax.ShapeDtypeStruct((M, 128), f32),
        ],
    )(sl4, tl4, leie4,
      wp["w4e"], wp["b4e"], wp["w2p"], wp["b2p"], wp["iem"],
      wp["wf4"], wp["bf4"], wp["wcfi"], wp["bcfi"], wp["wcfo"], wp["bcfo"])


def _edge_final4(sl4, tl4, leie4, wp):
    M = sl4.shape[0]
    BP = 800

    def body(sl_r, tl_r, leie_r, w4e, b4e, w2c, b2c, wc14, bc14, wc24, bc24,
             o_ref):
        e_in = jnp.concatenate(
            [sl_r[...].astype(bf16), tl_r[...].astype(bf16),
             leie_r[...].astype(bf16)], axis=1)
        h1 = jnp.maximum(_dot(e_in, w4e[...]) + b4e[...], 0.0)
        le_n = jnp.maximum(_dot(h1.astype(bf16), w2c[...]) + b2c[...], 0.0)
        hc = jnp.maximum(_dot(le_n.astype(bf16), wc14[...]) + bc14[...], 0.0)
        o_ref[...] = _dot(hc.astype(bf16), wc24[...]) + bc24[...]

    full = lambda a: pl.BlockSpec(a.shape, lambda i: (0,) * a.ndim)
    return pl.pallas_call(
        body,
        grid=(M // BP,),
        in_specs=[
            pl.BlockSpec((BP, 128), lambda i: (i, 0)),
            pl.BlockSpec((BP, 128), lambda i: (i, 0)),
            pl.BlockSpec((BP, 128), lambda i: (i, 0)),
        ] + [full(wp[k]) for k in
             ("w4e", "b4e", "w2c", "b2c", "wc14", "bc14", "wc24", "bc24")],
        out_specs=pl.BlockSpec((BP, 4), lambda i: (i, 0)),
        out_shape=jax.ShapeDtypeStruct((M, 4), f32),
    )(sl4, tl4, leie4,
      wp["w4e"], wp["b4e"], wp["w2c"], wp["b2c"],
      wp["wc14"], wp["bc14"], wp["wc24"], wp["bc24"])


def _node_mlp4(pfoA, pfiA, pfoB, pfiB, wnfi4, wnfo4, bn4, n):
    NP = n // 4

    def body(pfoA_r, pfiA_r, pfoB_r, pfiB_r, wi_ref, wo_ref, b_ref, o_ref):
        pfo = (pfoA_r[0, :NP] + pfoA_r[1, :NP]
               + pfoB_r[0, :NP] + pfoB_r[1, :NP]).astype(bf16)
        pfi = (pfiA_r[0, :NP] + pfiA_r[1, :NP]
               + pfiB_r[0, :NP] + pfiB_r[1, :NP]).astype(bf16)
        o_ref[...] = jnp.maximum(
            _dot(pfi, wi_ref[...]) + _dot(pfo, wo_ref[...]) + b_ref[...],
            0.0)

    return pl.pallas_call(
        body,
        out_shape=jax.ShapeDtypeStruct((NP, 128), f32),
    )(pfoA, pfiA, pfoB, pfiB, wnfi4.astype(bf16), wnfo4.astype(bf16), bn4)


KSUP = 4


def _sc_gather(lx_lin, row2d, col2d):
    nr = row2d.shape[0]
    n, D = lx_lin.shape
    assert nr % KSUP == 0
    nsup = nr // KSUP
    per_w = (nsup + NW - 1) // NW
    mesh = plsc.VectorSubcoreMesh(core_axis_name="c", subcore_axis_name="s")

    @functools.partial(
        pl.kernel,
        out_type=(jax.ShapeDtypeStruct((nr, CH, D), f32),
                  jax.ShapeDtypeStruct((nr, CH, D), f32)),
        mesh=mesh,
        scratch_types=[
            pltpu.VMEM((KSUP, CH), jnp.int32),
            pltpu.VMEM((KSUP, CH), jnp.int32),
            pltpu.VMEM((KSUP, CH, D), f32),
            pltpu.VMEM((KSUP, CH, D), f32),
            pltpu.SemaphoreType.DMA,
        ],
        compiler_params=pltpu.CompilerParams(use_tc_tiling_on_sc=False),
    )
    def k(lx_hbm, row_hbm, col_hbm, sl_hbm, tl_hbm, ri, ci, rbuf, cbuf, sem):
        wid = lax.axis_index("s") * NC + lax.axis_index("c")

        def body(j, carry):
            s = wid + j * NW

            @pl.when(s < nsup)
            def _():
                base = pl.multiple_of(s * KSUP, KSUP)
                pltpu.sync_copy(row_hbm.at[pl.ds(base, KSUP)], ri)
                pltpu.sync_copy(col_hbm.at[pl.ds(base, KSUP)], ci)
                cps = []
                for jj in range(KSUP):
                    cps.append(pltpu.async_copy(
                        lx_hbm.at[ri.at[jj]], rbuf.at[jj], sem))
                    cps.append(pltpu.async_copy(
                        lx_hbm.at[ci.at[jj]], cbuf.at[jj], sem))
                for cp in cps:
                    cp.wait()
                pltpu.sync_copy(rbuf, sl_hbm.at[pl.ds(base, KSUP)])
                pltpu.sync_copy(cbuf, tl_hbm.at[pl.ds(base, KSUP)])
            return carry

        lax.fori_loop(0, per_w, body, 0)

    return k(lx_lin, row2d, col2d)


def _sc_scatter2(fo3, fi3, ifo2d, ifi2d, zeros):
    nr, _, D = fo3.shape
    n_acc = zeros.shape[0]
    assert nr % KSUP == 0 and n_acc % NS == 0
    nsup = nr // KSUP
    per_w = (nsup + NW - 1) // NW
    rpt = n_acc // NS
    mesh = plsc.VectorSubcoreMesh(core_axis_name="c", subcore_axis_name="s")

    @functools.partial(
        pl.kernel,
        out_type=(jax.ShapeDtypeStruct((NC, n_acc, D), f32),
                  jax.ShapeDtypeStruct((NC, n_acc, D), f32)),
        mesh=mesh,
        scratch_types=[
            pltpu.VMEM_SHARED((n_acc, D), f32),
            pltpu.VMEM_SHARED((n_acc, D), f32),
            pltpu.VMEM((KSUP, CH), jnp.int32),
            pltpu.VMEM((KSUP, CH), jnp.int32),
            pltpu.VMEM((KSUP, CH, D), f32),
            pltpu.VMEM((KSUP, CH, D), f32),
        ],
        compiler_params=pltpu.CompilerParams(use_tc_tiling_on_sc=False),
    )
    def k(fo_hbm, fi_hbm, ifo_hbm, ifi_hbm, zeros_hbm, out_fo, out_fi,
          acc_fo, acc_fi, ri, si, dbuf, ebuf):
        cid = lax.axis_index("c")
        sid = lax.axis_index("s")
        wid = sid * NC + cid

        @pl.when(sid == 0)
        def _():
            pltpu.sync_copy(zeros_hbm, acc_fo)
            pltpu.sync_copy(zeros_hbm, acc_fi)

        plsc.subcore_barrier()

        def body(j, carry):
            s = wid + j * NW

            @pl.when(s < nsup)
            def _():
                base = pl.multiple_of(s * KSUP, KSUP)
                pltpu.sync_copy(ifo_hbm.at[pl.ds(base, KSUP)], ri)
                pltpu.sync_copy(ifi_hbm.at[pl.ds(base, KSUP)], si)
                pltpu.sync_copy(fo_hbm.at[pl.ds(base, KSUP)], dbuf)
                pltpu.sync_copy(fi_hbm.at[pl.ds(base, KSUP)], ebuf)
                for jj in range(KSUP):
                    pltpu.sync_copy(dbuf.at[jj], acc_fo.at[ri.at[jj]],
                                    add=True)
                    pltpu.sync_copy(ebuf.at[jj], acc_fi.at[si.at[jj]],
                                    add=True)
            return carry

        lax.fori_loop(0, per_w, body, 0)
        plsc.subcore_barrier()
        pltpu.sync_copy(acc_fo.at[pl.ds(sid * rpt, rpt)],
                        out_fo.at[cid].at[pl.ds(sid * rpt, rpt)])
        pltpu.sync_copy(acc_fi.at[pl.ds(sid * rpt, rpt)],
                        out_fi.at[cid].at[pl.ds(sid * rpt, rpt)])

    return k(fo3, fi3, ifo2d, ifi2d, zeros)


def kernel(x, edge_index, edge_attr, params):
    n = x.shape[0]
    E = edge_index.shape[1]
    row = edge_index[0]
    col = edge_index[1]
    p = params

    idx_fo = jnp.where(row < col, row, n).reshape(E // CH, CH)
    idx_fi = jnp.where(row > col, row, n).reshape(E // CH, CH)
    row2d = row.reshape(E // CH, CH)
    col2d = col.reshape(E // CH, CH)
    n_acc = n + 16

    (W1, b1), (W2, b2) = p["edge_model"]
    (Wfo1, bfo1), (Wfo2, bfo2) = p["flow_out"]
    (Wfi1, bfi1), (Wfi2, bfi2) = p["flow_in"]
    Wf1 = jnp.concatenate([Wfo1, Wfi1], axis=1)
    bf1c = jnp.concatenate([bfo1, bfi1])
    (Wn_, bn_) = p["node_mlp"][0]
    (Wc1, bc1), (Wc2, bc2) = p["classifier"]

    w4e = jnp.zeros((384, 320), f32)
    w2p = jnp.zeros((320, 128), f32)
    b2p = jnp.zeros((1, 128), f32)
    iem = jnp.zeros((1, 128), f32)
    wf4 = jnp.zeros((256, 448), f32)
    wcfi = jnp.zeros((448, 128), f32)
    wcfo = jnp.zeros((448, 128), f32)
    for g in range(4):
        w4e = w4e.at[32 * g:32 * g + 32, 80 * g:80 * g + 80].set(W1[:32])
        w4e = w4e.at[128 + 32 * g:128 + 32 * g + 32,
                     80 * g:80 * g + 80].set(W1[32:64])
        w4e = w4e.at[256 + 32 * g:256 + 32 * g + 32,
                     80 * g:80 * g + 80].set(W1[64:96])
        w2p = w2p.at[80 * g:80 * g + 80, 32 * g:32 * g + 16].set(W2)
        b2p = b2p.at[0, 32 * g:32 * g + 16].set(b2)
        iem = iem.at[0, 32 * g + 16:32 * g + 32].set(1.0)
        wf4 = wf4.at[32 * g:32 * g + 32, 112 * g:112 * g + 112].set(Wf1[:32])
        wf4 = wf4.at[128 + 32 * g:128 + 32 * g + 16,
                     112 * g:112 * g + 112].set(Wf1[32:48])
        wcfo = wcfo.at[112 * g:112 * g + 56,
                       32 * g:32 * g + 32].set(Wfo2)
        wcfi = wcfi.at[112 * g + 56:112 * g + 112,
                       32 * g:32 * g + 32].set(Wfi2)
    wp = {
        "w4e": w4e.astype(bf16), "b4e": jnp.tile(b1, 4).reshape(1, 320),
        "w2p": w2p.astype(bf16), "b2p": b2p, "iem": iem,
        "wf4": wf4.astype(bf16), "bf4": jnp.tile(bf1c, 4).reshape(1, 448),
        "wcfi": wcfi.astype(bf16), "bcfi": jnp.tile(bfi2, 4).reshape(1, 128),
        "wcfo": wcfo.astype(bf16), "bcfo": jnp.tile(bfo2, 4).reshape(1, 128),
        "w2c": _bd(W2, 4, 80, 16, (320, 64)).astype(bf16),
        "b2c": jnp.tile(b2, 4).reshape(1, 64),
        "wc14": _bd(Wc1, 4, 16, 8, (64, 32)).astype(bf16),
        "bc14": jnp.tile(bc1, 4).reshape(1, 32),
        "wc24": _bd(Wc2, 4, 8, 1, (32, 4)).astype(bf16),
        "bc24": jnp.tile(bc2, 4).reshape(1, 4),
    }
    wnfi4 = _bd(Wn_[:32], 4, 32, 32, (128, 128))
    wnfo4 = _bd(Wn_[32:], 4, 32, 32, (128, 128))
    bn4 = jnp.tile(bn_, 4).reshape(1, 128)

    (Wa1, ba1), (Wa2, ba2) = p["enc_edge"]
    wa1_8 = _bd(Wa1, 8, 16, 18, (128, 144))
    ba1_8 = jnp.tile(ba1, 8).reshape(1, 144)
    wa2_8 = jnp.zeros((144, 256), f32)
    ba2_8 = jnp.zeros((1, 256), f32)
    for g in range(8):
        wa2_8 = wa2_8.at[18 * g:18 * g + 18, 32 * g:32 * g + 16].set(Wa2)
        wa2_8 = wa2_8.at[18 * g:18 * g + 18,
                         32 * g + 16:32 * g + 32].set(Wa2)
        ba2_8 = ba2_8.at[0, 32 * g:32 * g + 16].set(ba2)
        ba2_8 = ba2_8.at[0, 32 * g + 16:32 * g + 32].set(ba2)

    (We1, be1), (We2, be2) = p["enc_node"]
    lx = _enc_node(x, We1, be1, We2, be2)
    lx4 = lx.reshape(n // 4, 128)
    leie8 = _enc_edge8(edge_attr.reshape(E // 8, 128),
                       wa1_8, ba1_8, wa2_8, ba2_8)
    leie4 = leie8.reshape(E // 4, 128)

    zeros = jnp.zeros((n_acc, 32), f32)

    SA = 1200
    EA = SA * CH
    halves = (
        dict(r2=row2d[:SA], c2=col2d[:SA], ifo=idx_fo[:SA],
             ifi=idx_fi[:SA], leie=leie4[:EA // 4]),
        dict(r2=row2d[SA:], c2=col2d[SA:], ifo=idx_fo[SA:],
             ifi=idx_fi[SA:], leie=leie4[EA // 4:]),
    )
    na4 = n_acc // 4
    for step in range(1, 5):
        lxlin = lx4.reshape(n, 32)
        for h in halves:
            h["sl"], h["tl"] = _sc_gather(lxlin, h["r2"], h["c2"])
        if step == 4:
            outs = []
            for h in halves:
                m = h["sl"].shape[0] * CH // 4
                outs.append(_edge_final4(h["sl"].reshape(m, 128),
                                         h["tl"].reshape(m, 128),
                                         h["leie"], wp))
            return jnp.concatenate(outs, axis=0).reshape(E, 1)
        parts = []
        for h in halves:
            m = h["sl"].shape[0] * CH // 4
            h["leie"], fi4, fo4 = _edge_step4(h["sl"].reshape(m, 128),
                                              h["tl"].reshape(m, 128),
                                              h["leie"], wp)
            nr_h = h["sl"].shape[0]
            parts.append(_sc_scatter2(fo4.reshape(nr_h, CH, 32),
                                      fi4.reshape(nr_h, CH, 32),
                                      h["ifo"], h["ifi"], zeros))
        (pfoA, pfiA), (pfoB, pfiB) = parts
        lx4 = _node_mlp4(pfoA.reshape(NC, na4, 128),
                         pfiA.reshape(NC, na4, 128),
                         pfoB.reshape(NC, na4, 128),
                         pfiB.reshape(NC, na4, 128),
                         wnfi4, wnfo4, bn4, n)
